# baseline (speedup 1.0000x reference)
import jax
import jax.numpy as jnp
from jax import lax
from jax.experimental import pallas as pl
from jax.experimental.pallas import tpu as pltpu
from jax.experimental.pallas import tpu_sc as plsc

N_NODES = 10000
N_EDGES = 320000
D_FEAT = 128

NC = 2
NS = 16
NW = NC * NS
CPT = D_FEAT // NW
N_PAD = 10240
EC = 2048
NECH = 160
E_PAD = NECH * EC


def _sc_body(xT_hbm, src_hbm, dst_hbm, w_hbm, out_hbm,
             xc, ac, er_src, er_dst, er_w, sem_e):
    cid = lax.axis_index("c")
    sid = lax.axis_index("s")
    gid = cid * NS + sid

    pltpu.sync_copy(xT_hbm.at[pl.ds(gid * CPT, CPT)], xc)

    def _z(i, carry):
        for c in range(CPT):
            ac[c, pl.ds(i * 16, 16)] = jnp.zeros((16,), jnp.float32)
        return carry
    lax.fori_loop(0, N_PAD // 16, _z, 0)

    def start_chunk(ci, b):
        pltpu.async_copy(src_hbm.at[ci], er_src.at[b], sem_e.at[b])
        pltpu.async_copy(dst_hbm.at[ci], er_dst.at[b], sem_e.at[b])
        pltpu.async_copy(w_hbm.at[ci], er_w.at[b], sem_e.at[b])

    def wait_chunk(ci, b):
        pltpu.make_async_copy(src_hbm.at[ci], er_src.at[b], sem_e.at[b]).wait()
        pltpu.make_async_copy(dst_hbm.at[ci], er_dst.at[b], sem_e.at[b]).wait()
        pltpu.make_async_copy(w_hbm.at[ci], er_w.at[b], sem_e.at[b]).wait()

    start_chunk(0, 0)
    start_chunk(1, 1)

    def _edge_group(g, carry):
        b = carry
        scaled = []
        dsts = []
        for u in range(2):
            sl = pl.ds((2 * g + u) * 16, 16)
            src_v = er_src[b, sl]
            dsts.append(er_dst[b, sl])
            w_v = er_w[b, sl]
            vals = [plsc.load_gather(xc.at[c], [src_v]) for c in range(CPT)]
            scaled.append([v * w_v for v in vals])
        for u in range(2):
            for c in range(CPT):
                plsc.addupdate_scatter(ac.at[c], [dsts[u]], scaled[u][c])
        return carry

    def _chunk(ci, b):
        wait_chunk(ci, b)
        lax.fori_loop(0, EC // 32, _edge_group, b)
        @pl.when(ci + 2 < NECH)
        def _():
            start_chunk(ci + 2, b)

    def _pair(i, carry):
        _chunk(2 * i, 0)
        _chunk(2 * i + 1, 1)
        return carry
    lax.fori_loop(0, NECH // 2, _pair, 0)

    pltpu.sync_copy(ac, out_hbm.at[pl.ds(gid * CPT, CPT)])


@jax.jit
def _sc_aggregate(xT, src_p, dst_p, w_p):
    mesh = plsc.VectorSubcoreMesh(core_axis_name="c", subcore_axis_name="s")
    f = pl.kernel(
        _sc_body,
        out_type=jax.ShapeDtypeStruct((D_FEAT, N_PAD), jnp.float32),
        mesh=mesh,
        scratch_types=[
            pltpu.VMEM((CPT, N_PAD), jnp.float32),
            pltpu.VMEM((CPT, N_PAD), jnp.float32),
            pltpu.VMEM((2, EC), jnp.int32),
            pltpu.VMEM((2, EC), jnp.int32),
            pltpu.VMEM((2, EC), jnp.float32),
            pltpu.SemaphoreType.DMA((2,)),
        ],
        compiler_params=pltpu.CompilerParams(use_tc_tiling_on_sc=False, needs_layout_passes=False),
    )
    return f(xT, src_p, dst_p, w_p)


def kernel(x, edge_index, edge_weight):
    src = edge_index[0]
    dst = edge_index[1]
    pad = E_PAD - N_EDGES
    src_p = jnp.concatenate([src, jnp.zeros((pad,), jnp.int32)]).reshape(
        NECH, EC)
    dst_p = jnp.concatenate([dst, jnp.zeros((pad,), jnp.int32)]).reshape(
        NECH, EC)
    w_p = jnp.concatenate(
        [edge_weight, jnp.zeros((pad,), jnp.float32)]).reshape(NECH, EC)
    xT = jnp.pad(x, ((0, N_PAD - N_NODES), (0, 0))).T
    outT = _sc_aggregate(xT, src_p, dst_p, w_p)
    return outT[:, :N_NODES].T


# speedup vs baseline: 1.0350x; 1.0350x over previous
import jax
import jax.numpy as jnp
from jax import lax
from jax.experimental import pallas as pl
from jax.experimental.pallas import tpu as pltpu
from jax.experimental.pallas import tpu_sc as plsc

N_NODES = 10000
N_EDGES = 320000
D_FEAT = 128

NC = 2
NS = 16
NW = NC * NS
CPT = D_FEAT // NW
N_PAD = 10240
EC = 2048
NECH = 160
E_PAD = NECH * EC


def _sc_body(xT_hbm, src_hbm, dst_hbm, w_hbm, out_hbm,
             xc, ac, er_src, er_dst, er_w, sem_e):
    cid = lax.axis_index("c")
    sid = lax.axis_index("s")
    gid = cid * NS + sid

    pltpu.sync_copy(xT_hbm.at[pl.ds(gid * CPT, CPT)], xc)

    def _z(i, carry):
        for c in range(CPT):
            ac[c, pl.ds(i * 16, 16)] = jnp.zeros((16,), jnp.float32)
        return carry
    lax.fori_loop(0, N_PAD // 16, _z, 0)

    def start_chunk(ci, b):
        pltpu.async_copy(src_hbm.at[ci], er_src.at[b], sem_e.at[b])
        pltpu.async_copy(dst_hbm.at[ci], er_dst.at[b], sem_e.at[b])
        pltpu.async_copy(w_hbm.at[ci], er_w.at[b], sem_e.at[b])

    def wait_chunk(ci, b):
        pltpu.make_async_copy(src_hbm.at[ci], er_src.at[b], sem_e.at[b]).wait()
        pltpu.make_async_copy(dst_hbm.at[ci], er_dst.at[b], sem_e.at[b]).wait()
        pltpu.make_async_copy(w_hbm.at[ci], er_w.at[b], sem_e.at[b]).wait()

    start_chunk(0, 0)
    start_chunk(1, 1)

    UNROLL = 4

    def _edge_group(g, carry):
        b = carry
        scaled = []
        dsts = []
        for u in range(UNROLL):
            sl = pl.ds((UNROLL * g + u) * 16, 16)
            src_v = er_src[b, sl]
            dsts.append(er_dst[b, sl])
            w_v = er_w[b, sl]
            vals = [plsc.load_gather(xc.at[c], [src_v]) for c in range(CPT)]
            scaled.append([v * w_v for v in vals])
        for u in range(UNROLL):
            for c in range(CPT):
                plsc.addupdate_scatter(ac.at[c], [dsts[u]], scaled[u][c])
        return carry

    def _chunk(ci, b):
        wait_chunk(ci, b)
        lax.fori_loop(0, EC // (16 * UNROLL), _edge_group, b)
        @pl.when(ci + 2 < NECH)
        def _():
            start_chunk(ci + 2, b)

    def _pair(i, carry):
        _chunk(2 * i, 0)
        _chunk(2 * i + 1, 1)
        return carry
    lax.fori_loop(0, NECH // 2, _pair, 0)

    pltpu.sync_copy(ac, out_hbm.at[pl.ds(gid * CPT, CPT)])


@jax.jit
def _sc_aggregate(xT, src_p, dst_p, w_p):
    mesh = plsc.VectorSubcoreMesh(core_axis_name="c", subcore_axis_name="s")
    f = pl.kernel(
        _sc_body,
        out_type=jax.ShapeDtypeStruct((D_FEAT, N_PAD), jnp.float32),
        mesh=mesh,
        scratch_types=[
            pltpu.VMEM((CPT, N_PAD), jnp.float32),
            pltpu.VMEM((CPT, N_PAD), jnp.float32),
            pltpu.VMEM((2, EC), jnp.int32),
            pltpu.VMEM((2, EC), jnp.int32),
            pltpu.VMEM((2, EC), jnp.float32),
            pltpu.SemaphoreType.DMA((2,)),
        ],
        compiler_params=pltpu.CompilerParams(use_tc_tiling_on_sc=False, needs_layout_passes=False),
    )
    return f(xT, src_p, dst_p, w_p)


def kernel(x, edge_index, edge_weight):
    src = edge_index[0]
    dst = edge_index[1]
    pad = E_PAD - N_EDGES
    src_p = jnp.concatenate([src, jnp.zeros((pad,), jnp.int32)]).reshape(
        NECH, EC)
    dst_p = jnp.concatenate([dst, jnp.zeros((pad,), jnp.int32)]).reshape(
        NECH, EC)
    w_p = jnp.concatenate(
        [edge_weight, jnp.zeros((pad,), jnp.float32)]).reshape(NECH, EC)
    xT = jnp.pad(x, ((0, N_PAD - N_NODES), (0, 0))).T
    outT = _sc_aggregate(xT, src_p, dst_p, w_p)
    return outT[:, :N_NODES].T


# speedup vs baseline: 1.0591x; 1.0234x over previous
import jax
import jax.numpy as jnp
from jax import lax
from jax.experimental import pallas as pl
from jax.experimental.pallas import tpu as pltpu
from jax.experimental.pallas import tpu_sc as plsc

N_NODES = 10000
N_EDGES = 320000
D_FEAT = 128

NC = 2
NS = 16
NW = NC * NS
CPT = D_FEAT // NW
N_PAD = 10240
EC = 2048
NECH = 160
E_PAD = NECH * EC


def _sc_body(xT_hbm, sd_hbm, w_hbm, out_hbm,
             xc, ac, er_sd, er_w, sem_e):
    cid = lax.axis_index("c")
    sid = lax.axis_index("s")
    gid = cid * NS + sid

    pltpu.sync_copy(xT_hbm.at[pl.ds(gid * CPT, CPT)], xc)

    def _z(i, carry):
        for c in range(CPT):
            ac[c, pl.ds(i * 16, 16)] = jnp.zeros((16,), jnp.float32)
        return carry
    lax.fori_loop(0, N_PAD // 16, _z, 0)

    def start_chunk(ci, b):
        pltpu.async_copy(sd_hbm.at[ci], er_sd.at[b], sem_e.at[b])
        pltpu.async_copy(w_hbm.at[ci], er_w.at[b], sem_e.at[b])

    def wait_chunk(ci, b):
        pltpu.make_async_copy(sd_hbm.at[ci], er_sd.at[b], sem_e.at[b]).wait()
        pltpu.make_async_copy(w_hbm.at[ci], er_w.at[b], sem_e.at[b]).wait()

    start_chunk(0, 0)
    start_chunk(1, 1)

    UNROLL = 4

    def _edge_group(g, carry):
        b = carry
        scaled = []
        dsts = []
        for u in range(UNROLL):
            sl = pl.ds((UNROLL * g + u) * 16, 16)
            sd_v = er_sd[b, sl]
            src_v = sd_v & jnp.int32(0xFFFF)
            dsts.append(lax.shift_right_logical(sd_v, jnp.int32(16)))
            w_v = er_w[b, sl]
            vals = [plsc.load_gather(xc.at[c], [src_v]) for c in range(CPT)]
            scaled.append([v * w_v for v in vals])
        for u in range(UNROLL):
            for c in range(CPT):
                plsc.addupdate_scatter(ac.at[c], [dsts[u]], scaled[u][c])
        return carry

    def _chunk(ci, b):
        wait_chunk(ci, b)
        lax.fori_loop(0, EC // (16 * UNROLL), _edge_group, b)
        @pl.when(ci + 2 < NECH)
        def _():
            start_chunk(ci + 2, b)

    def _pair(i, carry):
        _chunk(2 * i, 0)
        _chunk(2 * i + 1, 1)
        return carry
    lax.fori_loop(0, NECH // 2, _pair, 0)

    pltpu.sync_copy(ac, out_hbm.at[pl.ds(gid * CPT, CPT)])


@jax.jit
def _sc_aggregate(xT, sd_p, w_p):
    mesh = plsc.VectorSubcoreMesh(core_axis_name="c", subcore_axis_name="s")
    f = pl.kernel(
        _sc_body,
        out_type=jax.ShapeDtypeStruct((D_FEAT, N_PAD), jnp.float32),
        mesh=mesh,
        scratch_types=[
            pltpu.VMEM((CPT, N_PAD), jnp.float32),
            pltpu.VMEM((CPT, N_PAD), jnp.float32),
            pltpu.VMEM((2, EC), jnp.int32),
            pltpu.VMEM((2, EC), jnp.float32),
            pltpu.SemaphoreType.DMA((2,)),
        ],
        compiler_params=pltpu.CompilerParams(use_tc_tiling_on_sc=False, needs_layout_passes=False),
    )
    return f(xT, sd_p, w_p)


def kernel(x, edge_index, edge_weight):
    src = edge_index[0]
    dst = edge_index[1]
    pad = E_PAD - N_EDGES
    sd = src | (dst << 16)
    sd_p = jnp.concatenate([sd, jnp.zeros((pad,), jnp.int32)]).reshape(
        NECH, EC)
    w_p = jnp.concatenate(
        [edge_weight, jnp.zeros((pad,), jnp.float32)]).reshape(NECH, EC)
    xT = jnp.pad(x, ((0, N_PAD - N_NODES), (0, 0))).T
    outT = _sc_aggregate(xT, sd_p, w_p)
    return outT[:, :N_NODES].T
